# baseline (speedup 1.0000x reference)
import jax
import jax.numpy as jnp
from jax.experimental import pallas as pl

N = 524288
D = 128
H = 21
TASK = 3
BLK = 4096


def _mlp_body(x0_ref, x1_ref, x2_ref, w1_ref, b1_ref, w2_ref, b2_ref, out_ref):
    x = x0_ref[...]
    t = x1_ref[...]
    h1 = jnp.dot(x, w1_ref[...], preferred_element_type=jnp.float32)
    h1 = h1 + b1_ref[...]
    g = 0.5 * h1 * (1.0 + jax.lax.erf(h1 * 0.7071067811865476))
    h = jnp.sum(g * w2_ref[...], axis=1, keepdims=True) + b2_ref[...]
    out_ref[...] = jnp.where(t == TASK, h, x2_ref[...])


def kernel(x0, x1, x2, W1, b1, W2, b2):
    x1i = x1.astype(jnp.int32).reshape(N, 1)
    x2_new = pl.pallas_call(
        _mlp_body,
        grid=(N // BLK,),
        in_specs=[
            pl.BlockSpec((BLK, D), lambda i: (i, 0)),
            pl.BlockSpec((BLK, 1), lambda i: (i, 0)),
            pl.BlockSpec((BLK, 1), lambda i: (i, 0)),
            pl.BlockSpec((D, H), lambda i: (0, 0)),
            pl.BlockSpec((1, H), lambda i: (0, 0)),
            pl.BlockSpec((1, H), lambda i: (0, 0)),
            pl.BlockSpec((1, 1), lambda i: (0, 0)),
        ],
        out_specs=pl.BlockSpec((BLK, 1), lambda i: (i, 0)),
        out_shape=jax.ShapeDtypeStruct((N, 1), jnp.float32),
    )(x0, x1i, x2, W1, b1.reshape(1, H), W2.reshape(1, H), b2.reshape(1, 1))
    return (x0, x1, x2_new)


# speedup vs baseline: 2.7241x; 2.7241x over previous
import jax
import jax.numpy as jnp
from jax.experimental import pallas as pl

N = 524288
D = 128
H = 21
TASK = 3
BLK = 4096


def _mlp_body(x0_ref, x1_ref, x2_ref, w1_ref, b1_ref, w2_ref, b2_ref, out_ref):
    x = x0_ref[...]
    t = x1_ref[0]
    z = jax.lax.dot_general(
        w1_ref[...], x, (((0,), (1,)), ((), ())),
        preferred_element_type=jnp.float32,
    )
    z = z + b1_ref[...]
    g = 0.5 * z * (1.0 + jax.lax.erf(z * 0.7071067811865476))
    h = jnp.sum(g * w2_ref[...], axis=0, keepdims=True) + b2_ref[...]
    out_ref[0] = jnp.where(t == TASK, h, x2_ref[0])


def kernel(x0, x1, x2, W1, b1, W2, b2):
    x1i = x1.astype(jnp.int32).reshape(N // BLK, 1, BLK)
    x2r = x2.reshape(N // BLK, 1, BLK)
    x2_new = pl.pallas_call(
        _mlp_body,
        grid=(N // BLK,),
        in_specs=[
            pl.BlockSpec((BLK, D), lambda i: (i, 0)),
            pl.BlockSpec((1, 1, BLK), lambda i: (i, 0, 0)),
            pl.BlockSpec((1, 1, BLK), lambda i: (i, 0, 0)),
            pl.BlockSpec((D, H), lambda i: (0, 0)),
            pl.BlockSpec((H, 1), lambda i: (0, 0)),
            pl.BlockSpec((H, 1), lambda i: (0, 0)),
            pl.BlockSpec((1, 1), lambda i: (0, 0)),
        ],
        out_specs=pl.BlockSpec((1, 1, BLK), lambda i: (i, 0, 0)),
        out_shape=jax.ShapeDtypeStruct((N // BLK, 1, BLK), jnp.float32),
    )(x0, x1i, x2r, W1, b1.reshape(H, 1), W2, b2.reshape(1, 1))
    return (x0, x1, x2_new.reshape(N, 1))
